# Initial kernel scaffold; baseline (speedup 1.0000x reference)
import jax
import jax.numpy as jnp
from jax.experimental import pallas as pl


def kernel(params, edges_features, nodes, num_nodes, edges, num_edges):
    raise NotImplementedError("write your pallas kernel here")


# speedup vs baseline: 2.8381x; 2.8381x over previous
import functools

import jax
import jax.numpy as jnp
from jax import lax
from jax.experimental import pallas as pl
from jax.experimental.pallas import tpu as pltpu
from jax.experimental.pallas import tpu_sc as plsc

B = 20
NODES = 500
EPG = 16000
H = 128
NRBF = 50
NEMB = 119
N = B * NODES
E = B * EPG
LOG2 = 0.6931471805599453

NC = 2
NS = 16
NW = NC * NS
EPW = E // NW
C = 128
NFULL = EPW // C
TAIL = EPW - NFULL * C
RPT = (N // NS) // 8 * 8
RREM = N - RPT * NS


def _ssp(x):
    return jnp.maximum(x, 0.0) + jnp.log1p(jnp.exp(-jnp.abs(x))) - LOG2


def _sp(x):
    return jnp.maximum(x, 0.0) + jnp.log1p(jnp.exp(-jnp.abs(x)))


EBLK = 4000


def _gates_body(ef_ref, w1_ref, b1_ref, w2_ref, b2_ref, g0_ref, g1_ref, g2_ref):
    ef = ef_ref[:]
    mu = lax.broadcasted_iota(jnp.int32, (1, NRBF), 1).astype(jnp.float32) * 0.1
    rbf = jnp.exp(-50.0 * (ef - mu) ** 2)
    t = _ssp(jnp.dot(rbf, w1_ref[:], preferred_element_type=jnp.float32)
             + b1_ref[:])
    for i, out in enumerate((g0_ref, g1_ref, g2_ref)):
        out[:] = (jnp.dot(t[:, i * H:(i + 1) * H], w2_ref[i],
                          preferred_element_type=jnp.float32) + b2_ref[i])


def _gates_call(ef, w1c, b1c, w2s, b2s):
    return pl.pallas_call(
        _gates_body,
        grid=(E // EBLK,),
        in_specs=[
            pl.BlockSpec((EBLK, 1), lambda i: (i, 0)),
            pl.BlockSpec((NRBF, 3 * H), lambda i: (0, 0)),
            pl.BlockSpec((1, 3 * H), lambda i: (0, 0)),
            pl.BlockSpec((3, H, H), lambda i: (0, 0, 0)),
            pl.BlockSpec((3, 1, H), lambda i: (0, 0, 0)),
        ],
        out_specs=[pl.BlockSpec((EBLK, H), lambda i: (i, 0))] * 3,
        out_shape=[jax.ShapeDtypeStruct((E, H), jnp.float32)] * 3,
    )(ef, w1c, b1c, w2s, b2s)


def _emb_body(ids_ref, emb_ref, wn1, bn1, wn2, bn2, ns_ref, h_ref):
    ids = ids_ref[:]
    oh = (ids == lax.broadcasted_iota(jnp.int32, (N, NEMB), 1)
          ).astype(jnp.float32)
    ns = jnp.dot(oh, emb_ref[:], preferred_element_type=jnp.float32)
    ns_ref[:] = ns
    h_ref[:] = (jnp.dot(_ssp(jnp.dot(ns, wn1[:],
                                     preferred_element_type=jnp.float32)
                             + bn1[:]), wn2[:],
                        preferred_element_type=jnp.float32) + bn2[:])


def _emb_call(ids, emb, wn1, bn1, wn2, bn2):
    return pl.pallas_call(
        _emb_body,
        out_shape=[jax.ShapeDtypeStruct((N, H), jnp.float32)] * 2,
    )(ids, emb, wn1, bn1, wn2, bn2)


@functools.cache
def _build_msg_kernel():
  mesh = plsc.VectorSubcoreMesh(core_axis_name="c", subcore_axis_name="s",
                                num_cores=NC, num_subcores=NS)

  @functools.partial(
      pl.kernel,
      out_type=jax.ShapeDtypeStruct((NC, N, H), jnp.float32),
      mesh=mesh,
      scratch_types=[
        pltpu.VMEM((C,), jnp.int32),
        pltpu.VMEM((C,), jnp.int32),
        pltpu.VMEM((C, H), jnp.float32),
        pltpu.VMEM((C, H), jnp.float32),
        pltpu.VMEM((TAIL,), jnp.int32),
        pltpu.VMEM((TAIL,), jnp.int32),
        pltpu.VMEM((TAIL, H), jnp.float32),
        pltpu.VMEM((TAIL, H), jnp.float32),
          pltpu.VMEM_SHARED((N, H), jnp.float32),
          pltpu.SemaphoreType.DMA,
      ],
  )
  def _msg_kernel(h_hbm, g_hbm, src_hbm, dst_hbm, zero_hbm, out_hbm,
                  src_v, dst_v, gat_v, row_v,
                  src_t, dst_t, gat_t, row_t, acc, sem):
    cid = lax.axis_index("c")
    sid = lax.axis_index("s")
    wid = sid * NC + cid
    base = wid * EPW

    pltpu.sync_copy(zero_hbm.at[pl.ds(sid * RPT, RPT), :],
                    acc.at[pl.ds(sid * RPT, RPT), :])

    @pl.when(sid == NS - 1)
    def _():
        pltpu.sync_copy(zero_hbm.at[pl.ds(NS * RPT, RREM), :],
                        acc.at[pl.ds(NS * RPT, RREM), :])

    plsc.subcore_barrier()

    def do_chunk(off, csz, sb, db, gb, rb):
        off = pl.multiple_of(off, 8)
        pltpu.sync_copy(src_hbm.at[pl.ds(off, csz)], sb)
        pltpu.sync_copy(dst_hbm.at[pl.ds(off, csz)], db)
        pltpu.sync_copy(g_hbm.at[pl.ds(off, csz), :], gb)
        pltpu.async_copy(h_hbm.at[sb], rb, sem).wait()

        def mul_row(r, carry):
            for l in range(H // 16):
                s = pl.ds(l * 16, 16)
                rb[r, s] = rb[r, s] * gb[r, s]
            return carry

        lax.fori_loop(0, csz, mul_row, 0)
        pltpu.sync_copy(rb, acc.at[db], add=True)

    def body(j, carry):
        do_chunk(base + j * C, C, src_v, dst_v, gat_v, row_v)
        return carry

    lax.fori_loop(0, NFULL, body, 0)
    do_chunk(base + NFULL * C, TAIL, src_t, dst_t, gat_t, row_t)

    plsc.subcore_barrier()
    pltpu.sync_copy(acc.at[pl.ds(sid * RPT, RPT), :],
                    out_hbm.at[cid, pl.ds(sid * RPT, RPT), :])

    @pl.when(sid == NS - 1)
    def _():
        pltpu.sync_copy(acc.at[pl.ds(NS * RPT, RREM), :],
                        out_hbm.at[cid, pl.ds(NS * RPT, RREM), :])

  return _msg_kernel


def _upd_body(ns_ref, m_ref, ws1, bs1, ws2, bs2, wn1, bn1, wn2, bn2,
              nso_ref, h_ref):
    msg = m_ref[0] + m_ref[1]
    t = _ssp(jnp.dot(msg, ws1[:], preferred_element_type=jnp.float32) + bs1[:])
    ns2 = ns_ref[:] + jnp.dot(t, ws2[:],
                              preferred_element_type=jnp.float32) + bs2[:]
    nso_ref[:] = ns2
    h_ref[:] = (jnp.dot(_ssp(jnp.dot(ns2, wn1[:],
                                     preferred_element_type=jnp.float32)
                             + bn1[:]), wn2[:],
                        preferred_element_type=jnp.float32) + bn2[:])


def _upd_call(ns, msg, ws1, bs1, ws2, bs2, wn1, bn1, wn2, bn2):
    return pl.pallas_call(
        _upd_body,
        out_shape=[jax.ShapeDtypeStruct((N, H), jnp.float32)] * 2,
    )(ns, msg, ws1, bs1, ws2, bs2, wn1, bn1, wn2, bn2)


def _final_body(ns_ref, m_ref, ws1, bs1, ws2, bs2, wr1, br1, wr2, br2,
                wev, bev, out_ref):
    msg = m_ref[0] + m_ref[1]
    t = _ssp(jnp.dot(msg, ws1[:], preferred_element_type=jnp.float32) + bs1[:])
    ns2 = ns_ref[:] + jnp.dot(t, ws2[:],
                              preferred_element_type=jnp.float32) + bs2[:]
    r = _ssp(jnp.dot(ns2, wr1[:], preferred_element_type=jnp.float32) + br1[:])
    on = jnp.dot(r, wr2[:], preferred_element_type=jnp.float32) + br2[:]
    seg = (lax.broadcasted_iota(jnp.int32, (B, N), 1) // NODES
           == lax.broadcasted_iota(jnp.int32, (B, N), 0)).astype(jnp.float32)
    g = jnp.dot(seg, on, preferred_element_type=jnp.float32)
    ev = jnp.dot(g, wev[:], preferred_element_type=jnp.float32) + bev[:]
    out_ref[:] = jnp.concatenate(
        [ev[:, 0:1], _sp(ev[:, 1:2]), _sp(ev[:, 2:3]) + 1.0, _sp(ev[:, 3:4])],
        axis=1)


def _final_call(ns, msg, ws1, bs1, ws2, bs2, wr1, br1, wr2, br2, wev, bev):
    return pl.pallas_call(
        _final_body,
        out_shape=jax.ShapeDtypeStruct((B, 4), jnp.float32),
    )(ns, msg, ws1, bs1, ws2, bs2, wr1, br1, wr2, br2, wev, bev)


def _wb(p):
    return p["W"], p["b"][None, :]


def kernel(params, edges_features, nodes, num_nodes, edges, num_edges):
    inter = params["interactions"]
    ef = edges_features.reshape(E, 1)
    ids = nodes.reshape(N, 1)
    offs = (jnp.arange(B, dtype=jnp.int32) * NODES)[:, None]
    src = (edges[:, :, 0] + offs).reshape(E)
    dst = (edges[:, :, 1] + offs).reshape(E)
    zeros = jnp.zeros((N, H), jnp.float32)

    w1c = jnp.concatenate([it["edge1"]["W"] for it in inter], axis=1)
    b1c = jnp.concatenate([it["edge1"]["b"] for it in inter])[None, :]
    w2s = jnp.stack([it["edge2"]["W"] for it in inter])
    b2s = jnp.stack([it["edge2"]["b"][None, :] for it in inter])
    gates = _gates_call(ef, w1c, b1c, w2s, b2s)

    wn1, bn1 = _wb(inter[0]["node1"])
    wn2, bn2 = _wb(inter[0]["node2"])
    ns, h = _emb_call(ids, params["atom_emb"], wn1, bn1, wn2, bn2)

    msg_kernel = _build_msg_kernel()
    for i in range(len(inter)):
        msg = msg_kernel(h, gates[i], src, dst, zeros)
        ws1, bs1 = _wb(inter[i]["st1"])
        ws2, bs2 = _wb(inter[i]["st2"])
        if i + 1 < len(inter):
            wn1, bn1 = _wb(inter[i + 1]["node1"])
            wn2, bn2 = _wb(inter[i + 1]["node2"])
            ns, h = _upd_call(ns, msg, ws1, bs1, ws2, bs2, wn1, bn1, wn2, bn2)
        else:
            wr1, br1 = _wb(params["readout1"])
            wr2, br2 = _wb(params["readout2"])
            wev, bev = _wb(params["evidential"])
            out = _final_call(ns, msg, ws1, bs1, ws2, bs2,
                              wr1, br1, wr2, br2, wev, bev)
    return out


# speedup vs baseline: 4.8719x; 1.7166x over previous
import functools

import jax
import jax.numpy as jnp
from jax import lax
from jax.experimental import pallas as pl
from jax.experimental.pallas import tpu as pltpu
from jax.experimental.pallas import tpu_sc as plsc

B = 20
NODES = 500
EPG = 16000
H = 128
NRBF = 50
NEMB = 119
N = B * NODES
E = B * EPG
LOG2 = 0.6931471805599453

NC = 2
NS = 16
NW = NC * NS
EPW = E // NW
C = 128
NFULL = EPW // C
TAIL = EPW - NFULL * C
NL = N // NC
RPT = (NL // NS) // 8 * 8
RREM = NL - RPT * NS


def _ssp(x):
    return jnp.maximum(x, 0.0) + jnp.log1p(jnp.exp(-jnp.abs(x))) - LOG2


def _sp(x):
    return jnp.maximum(x, 0.0) + jnp.log1p(jnp.exp(-jnp.abs(x)))


EBLK = 4000


def _gates_body(ef_ref, w1_ref, b1_ref, w2_ref, b2_ref, g0_ref, g1_ref, g2_ref):
    ef = ef_ref[:]
    mu = lax.broadcasted_iota(jnp.int32, (1, NRBF), 1).astype(jnp.float32) * 0.1
    rbf = jnp.exp(-50.0 * (ef - mu) ** 2)
    t = _ssp(jnp.dot(rbf, w1_ref[:], preferred_element_type=jnp.float32)
             + b1_ref[:])
    for i, out in enumerate((g0_ref, g1_ref, g2_ref)):
        out[:] = (jnp.dot(t[:, i * H:(i + 1) * H], w2_ref[i],
                          preferred_element_type=jnp.float32) + b2_ref[i])


def _gates_call(ef, w1c, b1c, w2s, b2s):
    return pl.pallas_call(
        _gates_body,
        grid=(E // EBLK,),
        in_specs=[
            pl.BlockSpec((EBLK, 1), lambda i: (i, 0)),
            pl.BlockSpec((NRBF, 3 * H), lambda i: (0, 0)),
            pl.BlockSpec((1, 3 * H), lambda i: (0, 0)),
            pl.BlockSpec((3, H, H), lambda i: (0, 0, 0)),
            pl.BlockSpec((3, 1, H), lambda i: (0, 0, 0)),
        ],
        out_specs=[pl.BlockSpec((EBLK, H), lambda i: (i, 0))] * 3,
        out_shape=[jax.ShapeDtypeStruct((E, H), jnp.float32)] * 3,
    )(ef, w1c, b1c, w2s, b2s)


def _emb_body(ids_ref, emb_ref, wn1, bn1, wn2, bn2, ns_ref, h_ref):
    ids = ids_ref[:]
    oh = (ids == lax.broadcasted_iota(jnp.int32, (N, NEMB), 1)
          ).astype(jnp.float32)
    ns = jnp.dot(oh, emb_ref[:], preferred_element_type=jnp.float32)
    ns_ref[:] = ns
    h_ref[:] = (jnp.dot(_ssp(jnp.dot(ns, wn1[:],
                                     preferred_element_type=jnp.float32)
                             + bn1[:]), wn2[:],
                        preferred_element_type=jnp.float32) + bn2[:])


def _emb_call(ids, emb, wn1, bn1, wn2, bn2):
    return pl.pallas_call(
        _emb_body,
        out_shape=[jax.ShapeDtypeStruct((N, H), jnp.float32)] * 2,
    )(ids, emb, wn1, bn1, wn2, bn2)


NBUF = 2


@functools.cache
def _build_msg_kernel():
  mesh = plsc.VectorSubcoreMesh(core_axis_name="c", subcore_axis_name="s",
                                num_cores=NC, num_subcores=NS)

  @functools.partial(
      pl.kernel,
      out_type=jax.ShapeDtypeStruct((N, H), jnp.float32),
      mesh=mesh,
      scratch_types=[
          pltpu.VMEM((NFULL, C), jnp.int32),
          pltpu.VMEM((NFULL, C), jnp.int32),
          pltpu.VMEM((TAIL,), jnp.int32),
          pltpu.VMEM((TAIL,), jnp.int32),
          pltpu.VMEM((NBUF, C, H), jnp.float32),
          pltpu.VMEM((NBUF, C, H), jnp.float32),
          pltpu.VMEM((TAIL, H), jnp.float32),
          pltpu.VMEM((TAIL, H), jnp.float32),
          pltpu.VMEM_SHARED((NL, H), jnp.float32),
          pltpu.SemaphoreType.DMA,
          pltpu.SemaphoreType.DMA,
          pltpu.SemaphoreType.DMA,
          pltpu.SemaphoreType.DMA,
          pltpu.SemaphoreType.DMA,
      ],
  )
  def _msg_kernel(h_hbm, g_hbm, src2_hbm, dst2_hbm, srct_hbm, dstt_hbm,
                  zero_hbm, out_hbm,
                  srcm, dstm, srct, dstt, gb, rb, gbt, rbt, acc,
                  semg0, semg1, semt0, semt1, semx):
    semg = (semg0, semg1)
    semt = (semt0, semt1)
    cid = lax.axis_index("c")
    sid = lax.axis_index("s")
    wid = cid * NS + sid
    base = wid * EPW
    rbase = cid * NL

    pltpu.sync_copy(src2_hbm.at[wid], srcm)
    pltpu.sync_copy(dst2_hbm.at[wid], dstm)
    pltpu.sync_copy(srct_hbm.at[wid], srct)
    pltpu.sync_copy(dstt_hbm.at[wid], dstt)

    pltpu.sync_copy(zero_hbm.at[pl.ds(sid * RPT, RPT), :],
                    acc.at[pl.ds(sid * RPT, RPT), :])

    @pl.when(sid == NS - 1)
    def _():
        pltpu.sync_copy(zero_hbm.at[pl.ds(NS * RPT, RREM), :],
                        acc.at[pl.ds(NS * RPT, RREM), :])

    plsc.subcore_barrier()

    def issue(k, b):
        off = pl.multiple_of(base + k * C, 8)
        pltpu.async_copy(g_hbm.at[pl.ds(off, C), :], gb.at[b], semt[b])
        pltpu.async_copy(h_hbm.at[srcm.at[k]], rb.at[b], semg[b])

    def wait(b):
        pltpu.make_async_copy(g_hbm.at[pl.ds(0, C), :], gb.at[b],
                              semt[b]).wait()
        pltpu.make_async_copy(h_hbm.at[srcm.at[0]], rb.at[b],
                              semg[b]).wait()

    def mul(rbuf, gbuf, rows):
        def mul_row(r, carry):
            for l in range(H // 16):
                s = pl.ds(l * 16, 16)
                rbuf[r, s] = rbuf[r, s] * gbuf[r, s]
            return carry

        lax.fori_loop(0, rows, mul_row, 0)

    for b in range(NBUF):
        issue(b, b)

    def body(j, carry):
        for b in range(NBUF):
            k = NBUF * j + b
            wait(b)
            mul(rb.at[b], gb.at[b], C)
            pltpu.sync_copy(rb.at[b], acc.at[dstm.at[k]], add=True)

            @pl.when(j < NFULL // NBUF - 1)
            def _():
                issue(k + NBUF, b)

        return carry

    lax.fori_loop(0, NFULL // NBUF, body, 0)

    offt = pl.multiple_of(base + NFULL * C, 8)
    pltpu.sync_copy(g_hbm.at[pl.ds(offt, TAIL), :], gbt)
    pltpu.async_copy(h_hbm.at[srct], rbt, semx).wait()
    mul(rbt, gbt, TAIL)
    pltpu.sync_copy(rbt, acc.at[dstt], add=True)

    plsc.subcore_barrier()
    pltpu.sync_copy(acc.at[pl.ds(sid * RPT, RPT), :],
                    out_hbm.at[pl.ds(rbase + sid * RPT, RPT), :])

    @pl.when(sid == NS - 1)
    def _():
        pltpu.sync_copy(acc.at[pl.ds(NS * RPT, RREM), :],
                        out_hbm.at[pl.ds(rbase + NS * RPT, RREM), :])

  return _msg_kernel


def _upd_body(ns_ref, m_ref, ws1, bs1, ws2, bs2, wn1, bn1, wn2, bn2,
              nso_ref, h_ref):
    msg = m_ref[:]
    t = _ssp(jnp.dot(msg, ws1[:], preferred_element_type=jnp.float32) + bs1[:])
    ns2 = ns_ref[:] + jnp.dot(t, ws2[:],
                              preferred_element_type=jnp.float32) + bs2[:]
    nso_ref[:] = ns2
    h_ref[:] = (jnp.dot(_ssp(jnp.dot(ns2, wn1[:],
                                     preferred_element_type=jnp.float32)
                             + bn1[:]), wn2[:],
                        preferred_element_type=jnp.float32) + bn2[:])


def _upd_call(ns, msg, ws1, bs1, ws2, bs2, wn1, bn1, wn2, bn2):
    return pl.pallas_call(
        _upd_body,
        out_shape=[jax.ShapeDtypeStruct((N, H), jnp.float32)] * 2,
    )(ns, msg, ws1, bs1, ws2, bs2, wn1, bn1, wn2, bn2)


def _final_body(ns_ref, m_ref, ws1, bs1, ws2, bs2, wr1, br1, wr2, br2,
                wev, bev, out_ref):
    msg = m_ref[:]
    t = _ssp(jnp.dot(msg, ws1[:], preferred_element_type=jnp.float32) + bs1[:])
    ns2 = ns_ref[:] + jnp.dot(t, ws2[:],
                              preferred_element_type=jnp.float32) + bs2[:]
    r = _ssp(jnp.dot(ns2, wr1[:], preferred_element_type=jnp.float32) + br1[:])
    on = jnp.dot(r, wr2[:], preferred_element_type=jnp.float32) + br2[:]
    seg = (lax.broadcasted_iota(jnp.int32, (B, N), 1) // NODES
           == lax.broadcasted_iota(jnp.int32, (B, N), 0)).astype(jnp.float32)
    g = jnp.dot(seg, on, preferred_element_type=jnp.float32)
    ev = jnp.dot(g, wev[:], preferred_element_type=jnp.float32) + bev[:]
    out_ref[:] = jnp.concatenate(
        [ev[:, 0:1], _sp(ev[:, 1:2]), _sp(ev[:, 2:3]) + 1.0, _sp(ev[:, 3:4])],
        axis=1)


def _final_call(ns, msg, ws1, bs1, ws2, bs2, wr1, br1, wr2, br2, wev, bev):
    return pl.pallas_call(
        _final_body,
        out_shape=jax.ShapeDtypeStruct((B, 4), jnp.float32),
    )(ns, msg, ws1, bs1, ws2, bs2, wr1, br1, wr2, br2, wev, bev)


def _wb(p):
    return p["W"], p["b"][None, :]


def kernel(params, edges_features, nodes, num_nodes, edges, num_edges):
    inter = params["interactions"]
    ef = edges_features.reshape(E, 1)
    ids = nodes.reshape(N, 1)
    offs = (jnp.arange(B, dtype=jnp.int32) * NODES)[:, None]
    loff = offs - jnp.where(jnp.arange(B)[:, None] >= B // NC, NL, 0)
    src = (edges[:, :, 0] + offs).reshape(NW, EPW)
    dst = (edges[:, :, 1] + loff).reshape(NW, EPW)
    src2 = src[:, :NFULL * C].reshape(NW, NFULL, C)
    srct = src[:, NFULL * C:]
    dst2 = dst[:, :NFULL * C].reshape(NW, NFULL, C)
    dstt = dst[:, NFULL * C:]
    zeros = jnp.zeros((NL, H), jnp.float32)

    w1c = jnp.concatenate([it["edge1"]["W"] for it in inter], axis=1)
    b1c = jnp.concatenate([it["edge1"]["b"] for it in inter])[None, :]
    w2s = jnp.stack([it["edge2"]["W"] for it in inter])
    b2s = jnp.stack([it["edge2"]["b"][None, :] for it in inter])
    gates = _gates_call(ef, w1c, b1c, w2s, b2s)

    wn1, bn1 = _wb(inter[0]["node1"])
    wn2, bn2 = _wb(inter[0]["node2"])
    ns, h = _emb_call(ids, params["atom_emb"], wn1, bn1, wn2, bn2)

    msg_kernel = _build_msg_kernel()
    for i in range(len(inter)):
        msg = msg_kernel(h, gates[i], src2, dst2, srct, dstt, zeros)
        ws1, bs1 = _wb(inter[i]["st1"])
        ws2, bs2 = _wb(inter[i]["st2"])
        if i + 1 < len(inter):
            wn1, bn1 = _wb(inter[i + 1]["node1"])
            wn2, bn2 = _wb(inter[i + 1]["node2"])
            ns, h = _upd_call(ns, msg, ws1, bs1, ws2, bs2, wn1, bn1, wn2, bn2)
        else:
            wr1, br1 = _wb(params["readout1"])
            wr2, br2 = _wb(params["readout2"])
            wev, bev = _wb(params["evidential"])
            out = _final_call(ns, msg, ws1, bs1, ws2, bs2,
                              wr1, br1, wr2, br2, wev, bev)
    return out
